# baseline (speedup 1.0000x reference)
import functools

import jax
import jax.numpy as jnp
from jax import lax
from jax.experimental import pallas as pl
from jax.experimental.pallas import tpu as pltpu
from jax.experimental.pallas import tpu_sc as plsc

TOP_K = 2
EXPLORATION_BONUS = 0.1


def _tc_body(x_ref, w1t_ref, b1_ref, w2_ref, b2_ref, s1t_ref, s1b_ref,
             s2t_ref, s2b_ref, we_ref, be_ref, sc_ref, pr_ref,
             w1_scr, s1_scr, s2_scr, web_scr):
    @pl.when(pl.program_id(0) == 0)
    def _prep():
        w1_scr[...] = w1t_ref[...].T
        s1_scr[...] = s1t_ref[...].T
        s2_scr[...] = s2t_ref[...].T
        web_scr[...] = we_ref[...].T.astype(jnp.bfloat16)

    xt = x_ref[...]
    xb = xt.astype(jnp.bfloat16)
    h = jnp.maximum(
        jnp.dot(xt, w1_scr[...], preferred_element_type=jnp.float32)
        + b1_ref[...].reshape(1, -1), 0.0)
    ctx = jnp.tanh(
        jnp.dot(h, w2_ref[...], preferred_element_type=jnp.float32)
        + b2_ref[...].reshape(1, -1))
    sh = jnp.maximum(
        jnp.dot(ctx, s1_scr[...], preferred_element_type=jnp.float32)
        + s1b_ref[...].reshape(1, -1), 0.0)
    scores = (jnp.dot(sh, s2_scr[...], preferred_element_type=jnp.float32)
              + s2b_ref[...].reshape(1, -1) + EXPLORATION_BONUS)
    preds = (jnp.dot(xb, web_scr[...], preferred_element_type=jnp.float32)
             + be_ref[...].reshape(1, -1))
    sc_ref[...] = scores.T
    pr_ref[...] = preds.T


def _sc_route(scores_hbm, preds_hbm, pred_hbm, rw_hbm,
              sc_v, pr_v, po_v, w1_v, w2_v, *, n_experts, chunk, lanes,
              num_cores):
    wid = lax.axis_index("s") * num_cores + lax.axis_index("c")
    base = wid * chunk
    pltpu.sync_copy(scores_hbm.at[:, pl.ds(base, chunk)], sc_v)
    pltpu.sync_copy(preds_hbm.at[:, pl.ds(base, chunk)], pr_v)

    def body(g, carry):
        col0 = g * lanes
        m1 = sc_v[0, pl.ds(col0, lanes)]
        i1 = jnp.zeros((lanes,), jnp.float32)
        m2 = jnp.full((lanes,), -jnp.inf, jnp.float32)
        i2 = jnp.zeros((lanes,), jnp.float32)
        for e in range(1, n_experts):
            s = sc_v[e, pl.ds(col0, lanes)]
            gt1 = s > m1
            gt2 = s > m2
            i2 = jnp.where(gt1, i1, jnp.where(gt2, float(e), i2))
            m2 = jnp.where(gt1, m1, jnp.where(gt2, s, m2))
            i1 = jnp.where(gt1, float(e), i1)
            m1 = jnp.where(gt1, s, m1)
        e2 = jnp.exp(m2 - m1)
        den = 1.0 + e2
        w1 = 1.0 / den
        w2 = e2 / den
        cols = col0 + lax.iota(jnp.int32, lanes)
        p1 = plsc.load_gather(pr_v, [i1.astype(jnp.int32), cols])
        p2 = plsc.load_gather(pr_v, [i2.astype(jnp.int32), cols])
        po_v[pl.ds(col0, lanes)] = w1 * p1 + w2 * p2
        w1_v[pl.ds(col0, lanes)] = w1
        w2_v[pl.ds(col0, lanes)] = w2
        return carry

    lax.fori_loop(0, chunk // lanes, body, 0)
    pltpu.sync_copy(po_v, pred_hbm.at[pl.ds(base, chunk)])
    pltpu.sync_copy(w1_v, rw_hbm.at[0, pl.ds(base, chunk)])
    pltpu.sync_copy(w2_v, rw_hbm.at[1, pl.ds(base, chunk)])


def kernel(x, W1, b1, W2, b2, S1, s1, S2, s2, We, be):
    n, d = x.shape
    e = S2.shape[1]
    hid1 = W1.shape[1]
    ctxd = W2.shape[1]
    hid2 = S1.shape[1]

    tile = 4096
    grid = n // tile
    c1 = lambda i: (0,)
    c2 = lambda i: (0, 0)

    scores_t, preds_t = pl.pallas_call(
        _tc_body,
        grid=(grid,),
        in_specs=[
            pl.BlockSpec((tile, d), lambda i: (i, 0)),
            pl.BlockSpec((hid1, d), c2),
            pl.BlockSpec((hid1,), c1),
            pl.BlockSpec((hid1, ctxd), c2),
            pl.BlockSpec((ctxd,), c1),
            pl.BlockSpec((hid2, ctxd), c2),
            pl.BlockSpec((hid2,), c1),
            pl.BlockSpec((e, hid2), c2),
            pl.BlockSpec((e,), c1),
            pl.BlockSpec((e, d), c2),
            pl.BlockSpec((e,), c1),
        ],
        out_specs=[
            pl.BlockSpec((e, tile), lambda i: (0, i)),
            pl.BlockSpec((e, tile), lambda i: (0, i)),
        ],
        out_shape=[
            jax.ShapeDtypeStruct((e, n), jnp.float32),
            jax.ShapeDtypeStruct((e, n), jnp.float32),
        ],
        scratch_shapes=[
            pltpu.VMEM((d, hid1), jnp.float32),
            pltpu.VMEM((ctxd, hid2), jnp.float32),
            pltpu.VMEM((hid2, e), jnp.float32),
            pltpu.VMEM((d, e), jnp.bfloat16),
        ],
    )(x, W1.T, b1, W2, b2, S1.T, s1, S2.T, s2, We[:, :, 0], be.reshape(e))

    info = plsc.get_sparse_core_info()
    nw = info.num_cores * info.num_subcores
    lanes = info.num_lanes
    chunk = n // nw
    mesh = plsc.VectorSubcoreMesh(core_axis_name="c", subcore_axis_name="s")

    route = functools.partial(
        pl.kernel,
        mesh=mesh,
        compiler_params=pltpu.CompilerParams(needs_layout_passes=False),
        out_type=[
            jax.ShapeDtypeStruct((n,), jnp.float32),
            jax.ShapeDtypeStruct((TOP_K, n), jnp.float32),
        ],
        scratch_types=[
            pltpu.VMEM((e, chunk), jnp.float32),
            pltpu.VMEM((e, chunk), jnp.float32),
            pltpu.VMEM((chunk,), jnp.float32),
            pltpu.VMEM((chunk,), jnp.float32),
            pltpu.VMEM((chunk,), jnp.float32),
        ],
    )(functools.partial(_sc_route, n_experts=e, chunk=chunk, lanes=lanes,
                        num_cores=info.num_cores))

    pred, rw = route(scores_t, preds_t)
    return (pred.reshape(n, 1), rw.T)


# speedup vs baseline: 1.3472x; 1.3472x over previous
import functools

import jax
import jax.numpy as jnp
from jax.experimental import pallas as pl
from jax.experimental.pallas import tpu as pltpu

TOP_K = 2
EXPLORATION_BONUS = 0.1


def _body(x_ref, w1t_ref, b1_ref, w2_ref, b2_ref, s1t_ref, s1b_ref,
          s2t_ref, s2b_ref, we_ref, be_ref, pred_ref, rw_ref,
          w1_scr, s1_scr, s2_scr, web_scr, *, n_experts):
    @pl.when(pl.program_id(0) == 0)
    def _prep():
        w1_scr[...] = w1t_ref[...].T
        s1_scr[...] = s1t_ref[...].T
        s2_scr[...] = s2t_ref[...].T
        web_scr[...] = we_ref[...].T.astype(jnp.bfloat16)

    xt = x_ref[...]
    xb = xt.astype(jnp.bfloat16)
    h = jnp.maximum(
        jnp.dot(xt, w1_scr[...], preferred_element_type=jnp.float32)
        + b1_ref[...].reshape(1, -1), 0.0)
    ctx = jnp.tanh(
        jnp.dot(h, w2_ref[...], preferred_element_type=jnp.float32)
        + b2_ref[...].reshape(1, -1))
    sh = jnp.maximum(
        jnp.dot(ctx, s1_scr[...], preferred_element_type=jnp.float32)
        + s1b_ref[...].reshape(1, -1), 0.0)
    scores = (jnp.dot(sh, s2_scr[...], preferred_element_type=jnp.float32)
              + s2b_ref[...].reshape(1, -1) + EXPLORATION_BONUS)

    preds = (jnp.dot(xb, web_scr[...], preferred_element_type=jnp.float32)
             + be_ref[...].reshape(1, -1))

    scores_t = scores.T
    preds_t = preds.T

    eidx = jax.lax.broadcasted_iota(jnp.int32, scores_t.shape, 0).astype(
        jnp.float32)
    m1 = jnp.max(scores_t, axis=0, keepdims=True)
    i1 = jnp.min(jnp.where(scores_t == m1, eidx, float(n_experts)), axis=0,
                 keepdims=True)
    masked = jnp.where(eidx == i1, -jnp.inf, scores_t)
    m2 = jnp.max(masked, axis=0, keepdims=True)
    i2 = jnp.min(jnp.where(masked == m2, eidx, float(n_experts)), axis=0,
                 keepdims=True)

    e2 = jnp.exp(m2 - m1)
    denom = 1.0 + e2
    w1v = 1.0 / denom
    w2v = e2 / denom

    sel = jnp.where(eidx == i1, w1v, 0.0) + jnp.where(eidx == i2, w2v, 0.0)
    pred_ref[...] = jnp.sum(sel * preds_t, axis=0, keepdims=True)
    rw_ref[...] = jnp.concatenate([w1v, w2v], axis=0)


def kernel(x, W1, b1, W2, b2, S1, s1, S2, s2, We, be):
    n, d = x.shape
    e = S2.shape[1]
    hid1 = W1.shape[1]
    ctxd = W2.shape[1]
    hid2 = S1.shape[1]

    tile = 4096
    grid = n // tile
    c1 = lambda i: (0,)
    c2 = lambda i: (0, 0)

    preds, rw = pl.pallas_call(
        functools.partial(_body, n_experts=e),
        grid=(grid,),
        in_specs=[
            pl.BlockSpec((tile, d), lambda i: (i, 0)),
            pl.BlockSpec((hid1, d), c2),
            pl.BlockSpec((hid1,), c1),
            pl.BlockSpec((hid1, ctxd), c2),
            pl.BlockSpec((ctxd,), c1),
            pl.BlockSpec((hid2, ctxd), c2),
            pl.BlockSpec((hid2,), c1),
            pl.BlockSpec((e, hid2), c2),
            pl.BlockSpec((e,), c1),
            pl.BlockSpec((e, d), c2),
            pl.BlockSpec((e,), c1),
        ],
        out_specs=[
            pl.BlockSpec((1, tile), lambda i: (0, i)),
            pl.BlockSpec((TOP_K, tile), lambda i: (0, i)),
        ],
        out_shape=[
            jax.ShapeDtypeStruct((1, n), jnp.float32),
            jax.ShapeDtypeStruct((TOP_K, n), jnp.float32),
        ],
        scratch_shapes=[
            pltpu.VMEM((d, hid1), jnp.float32),
            pltpu.VMEM((ctxd, hid2), jnp.float32),
            pltpu.VMEM((hid2, e), jnp.float32),
            pltpu.VMEM((d, e), jnp.bfloat16),
        ],
    )(x, W1.T, b1, W2, b2, S1.T, s1, S2.T, s2, We[:, :, 0], be.reshape(e))
    return (preds.reshape(n, 1), rw.T)


# speedup vs baseline: 1.3531x; 1.0043x over previous
import functools

import jax
import jax.numpy as jnp
from jax.experimental import pallas as pl
from jax.experimental.pallas import tpu as pltpu

TOP_K = 2
EXPLORATION_BONUS = 0.1


def _body(x_ref, w1t_ref, b1_ref, w2_ref, b2_ref, s1t_ref, s1b_ref,
          s2t_ref, s2b_ref, we_ref, be_ref, pred_ref, rw_ref,
          w1_scr, s1_scr, s2_scr, web_scr, *, n_experts):
    @pl.when(pl.program_id(0) == 0)
    def _prep():
        w1_scr[...] = w1t_ref[...].T
        s1_scr[...] = s1t_ref[...].T
        s2_scr[...] = s2t_ref[...].T
        web_scr[...] = we_ref[...].T

    xt = x_ref[...]
    h = jnp.maximum(
        jnp.dot(xt, w1_scr[...], preferred_element_type=jnp.float32)
        + b1_ref[...].reshape(1, -1), 0.0)
    ctx = jnp.tanh(
        jnp.dot(h, w2_ref[...], preferred_element_type=jnp.float32)
        + b2_ref[...].reshape(1, -1))
    sh = jnp.maximum(
        jnp.dot(ctx, s1_scr[...], preferred_element_type=jnp.float32)
        + s1b_ref[...].reshape(1, -1), 0.0)
    scores = (jnp.dot(sh, s2_scr[...], preferred_element_type=jnp.float32)
              + s2b_ref[...].reshape(1, -1) + EXPLORATION_BONUS)

    preds = (jnp.dot(xt, web_scr[...], preferred_element_type=jnp.float32)
             + be_ref[...].reshape(1, -1))

    scores_t = scores.T
    preds_t = preds.T

    eidx = jax.lax.broadcasted_iota(jnp.int32, scores_t.shape, 0).astype(
        jnp.float32)
    m1 = jnp.max(scores_t, axis=0, keepdims=True)
    i1 = jnp.min(jnp.where(scores_t == m1, eidx, float(n_experts)), axis=0,
                 keepdims=True)
    masked = jnp.where(eidx == i1, -jnp.inf, scores_t)
    m2 = jnp.max(masked, axis=0, keepdims=True)
    i2 = jnp.min(jnp.where(masked == m2, eidx, float(n_experts)), axis=0,
                 keepdims=True)

    e2 = jnp.exp(m2 - m1)
    denom = 1.0 + e2
    w1v = 1.0 / denom
    w2v = e2 / denom

    sel = jnp.where(eidx == i1, w1v, 0.0) + jnp.where(eidx == i2, w2v, 0.0)
    pred_ref[...] = jnp.sum(sel * preds_t, axis=0, keepdims=True)
    rw_ref[...] = jnp.concatenate([w1v, w2v], axis=0)


def kernel(x, W1, b1, W2, b2, S1, s1, S2, s2, We, be):
    n, d = x.shape
    e = S2.shape[1]
    hid1 = W1.shape[1]
    ctxd = W2.shape[1]
    hid2 = S1.shape[1]

    tile = 4096
    grid = n // tile
    c1 = lambda i: (0,)
    c2 = lambda i: (0, 0)

    preds, rw = pl.pallas_call(
        functools.partial(_body, n_experts=e),
        grid=(grid,),
        in_specs=[
            pl.BlockSpec((tile, d), lambda i: (i, 0)),
            pl.BlockSpec((hid1, d), c2),
            pl.BlockSpec((hid1,), c1),
            pl.BlockSpec((hid1, ctxd), c2),
            pl.BlockSpec((ctxd,), c1),
            pl.BlockSpec((hid2, ctxd), c2),
            pl.BlockSpec((hid2,), c1),
            pl.BlockSpec((e, hid2), c2),
            pl.BlockSpec((e,), c1),
            pl.BlockSpec((e, d), c2),
            pl.BlockSpec((e,), c1),
        ],
        out_specs=[
            pl.BlockSpec((1, tile), lambda i: (0, i)),
            pl.BlockSpec((TOP_K, tile), lambda i: (0, i)),
        ],
        out_shape=[
            jax.ShapeDtypeStruct((1, n), jnp.float32),
            jax.ShapeDtypeStruct((TOP_K, n), jnp.float32),
        ],
        scratch_shapes=[
            pltpu.VMEM((d, hid1), jnp.float32),
            pltpu.VMEM((ctxd, hid2), jnp.float32),
            pltpu.VMEM((hid2, e), jnp.float32),
            pltpu.VMEM((d, e), jnp.float32),
        ],
    )(x, W1.T, b1, W2, b2, S1.T, s1, S2.T, s2, We[:, :, 0], be.reshape(e))
    return (preds.reshape(n, 1), rw.T)


# speedup vs baseline: 1.5427x; 1.1402x over previous
import functools

import jax
import jax.numpy as jnp
from jax.experimental import pallas as pl
from jax.experimental.pallas import tpu as pltpu

TOP_K = 2
EXPLORATION_BONUS = 0.1


def _body(x_ref, w1t_ref, b1_ref, w2_ref, b2_ref, s1t_ref, s1b_ref,
          s2t_ref, s2b_ref, we_ref, be_ref, pred_ref, rw_ref,
          w1_scr, s1_scr, s2_scr, web_scr, hacc, pacc,
          *, n_experts, n_dchunks):
    j = pl.program_id(1)
    dc = x_ref.shape[1]
    joff = j * dc

    @pl.when(pl.program_id(0) == 0)
    def _prep():
        w1_scr[pl.ds(joff, dc), :] = w1t_ref[...].T
        web_scr[pl.ds(joff, dc), :] = we_ref[...].T

        @pl.when(j == 0)
        def _prep_once():
            s1_scr[...] = s1t_ref[...].T
            s2_scr[...] = s2t_ref[...].T

    xt = x_ref[...]
    h_part = jnp.dot(xt, w1_scr[pl.ds(joff, dc), :],
                     preferred_element_type=jnp.float32)
    p_part = jnp.dot(xt, web_scr[pl.ds(joff, dc), :],
                     preferred_element_type=jnp.float32)

    @pl.when(j == 0)
    def _init():
        hacc[...] = h_part
        pacc[...] = p_part

    @pl.when(j > 0)
    def _acc():
        hacc[...] += h_part
        pacc[...] += p_part

    @pl.when(j == n_dchunks - 1)
    def _finish():
        h = jnp.maximum(hacc[...] + b1_ref[...].reshape(1, -1), 0.0)
        ctx = jnp.tanh(
            jnp.dot(h, w2_ref[...], preferred_element_type=jnp.float32)
            + b2_ref[...].reshape(1, -1))
        sh = jnp.maximum(
            jnp.dot(ctx, s1_scr[...], preferred_element_type=jnp.float32)
            + s1b_ref[...].reshape(1, -1), 0.0)
        scores = (jnp.dot(sh, s2_scr[...], preferred_element_type=jnp.float32)
                  + s2b_ref[...].reshape(1, -1) + EXPLORATION_BONUS)
        preds = pacc[...] + be_ref[...].reshape(1, -1)

        scores_t = scores.T
        preds_t = preds.T

        eidx = jax.lax.broadcasted_iota(jnp.int32, scores_t.shape, 0).astype(
            jnp.float32)
        m1 = jnp.max(scores_t, axis=0, keepdims=True)
        i1 = jnp.min(jnp.where(scores_t == m1, eidx, float(n_experts)),
                     axis=0, keepdims=True)
        masked = jnp.where(eidx == i1, -jnp.inf, scores_t)
        m2 = jnp.max(masked, axis=0, keepdims=True)
        i2 = jnp.min(jnp.where(masked == m2, eidx, float(n_experts)),
                     axis=0, keepdims=True)

        e2 = jnp.exp(m2 - m1)
        denom = 1.0 + e2
        w1v = 1.0 / denom
        w2v = e2 / denom

        sel = (jnp.where(eidx == i1, w1v, 0.0)
               + jnp.where(eidx == i2, w2v, 0.0))
        pred_ref[...] = jnp.sum(sel * preds_t, axis=0, keepdims=True)
        rw_ref[...] = jnp.concatenate([w1v, w2v], axis=0)


def kernel(x, W1, b1, W2, b2, S1, s1, S2, s2, We, be):
    n, d = x.shape
    e = S2.shape[1]
    hid1 = W1.shape[1]
    ctxd = W2.shape[1]
    hid2 = S1.shape[1]

    tile = 4096
    dchunks = 2
    dc = d // dchunks
    grid = (n // tile, dchunks)
    c1 = lambda i, j: (0,)
    c2 = lambda i, j: (0, 0)

    preds, rw = pl.pallas_call(
        functools.partial(_body, n_experts=e, n_dchunks=dchunks),
        grid=grid,
        in_specs=[
            pl.BlockSpec((tile, dc), lambda i, j: (i, j)),
            pl.BlockSpec((hid1, dc), lambda i, j: (0, j)),
            pl.BlockSpec((hid1,), c1),
            pl.BlockSpec((hid1, ctxd), c2),
            pl.BlockSpec((ctxd,), c1),
            pl.BlockSpec((hid2, ctxd), c2),
            pl.BlockSpec((hid2,), c1),
            pl.BlockSpec((e, hid2), c2),
            pl.BlockSpec((e,), c1),
            pl.BlockSpec((e, dc), lambda i, j: (0, j)),
            pl.BlockSpec((e,), c1),
        ],
        out_specs=[
            pl.BlockSpec((1, tile), lambda i, j: (0, i)),
            pl.BlockSpec((TOP_K, tile), lambda i, j: (0, i)),
        ],
        out_shape=[
            jax.ShapeDtypeStruct((1, n), jnp.float32),
            jax.ShapeDtypeStruct((TOP_K, n), jnp.float32),
        ],
        scratch_shapes=[
            pltpu.VMEM((d, hid1), jnp.float32),
            pltpu.VMEM((ctxd, hid2), jnp.float32),
            pltpu.VMEM((hid2, e), jnp.float32),
            pltpu.VMEM((d, e), jnp.float32),
            pltpu.VMEM((tile, hid1), jnp.float32),
            pltpu.VMEM((tile, e), jnp.float32),
        ],
    )(x, W1.T, b1, W2, b2, S1.T, s1, S2.T, s2, We[:, :, 0], be.reshape(e))
    return (preds.reshape(n, 1), rw.T)
